# baseline (speedup 1.0000x reference)
import functools

import jax
import jax.numpy as jnp
from jax import lax
from jax.experimental import pallas as pl
from jax.experimental.pallas import tpu as pltpu
from jax.experimental.pallas import tpu_sc as plsc


def _make_gather(Bb: int, S: int, D: int, vocab: int):
    info = plsc.get_sparse_core_info()
    nw = info.num_cores * info.num_subcores
    b_per_w = (Bb * S) // nw
    assert S % b_per_w == 0
    wpr = S // b_per_w
    ch = 8
    n_ch = b_per_w // ch
    mesh = plsc.VectorSubcoreMesh(core_axis_name="c", subcore_axis_name="s")

    @functools.partial(
        pl.kernel,
        mesh=mesh,
        out_type=(
            jax.ShapeDtypeStruct((Bb, S, D), jnp.float32),
            jax.ShapeDtypeStruct((Bb, S), jnp.int32),
        ),
        scratch_types=[
            pltpu.VMEM((b_per_w,), jnp.int32),
            pltpu.VMEM((b_per_w,), jnp.int32),
            pltpu.VMEM((3, 4 * ch, D), jnp.float32),
        ] + [pltpu.SemaphoreType.DMA] * (9 + 3 + 1),
    )
    def gather(table_hbm, idx_hbm,
               out_hbm, idx_out,
               idx_v, idx_c, rows_v, *sems):
        gsem, wsem = sems[:9], sems[9:12]
        s_wi = sems[12]
        wid = lax.axis_index("s") * info.num_cores + lax.axis_index("c")
        row = wid // wpr
        col = (wid % wpr) * b_per_w
        span = pl.ds(col, b_per_w)

        pltpu.sync_copy(idx_hbm.at[row, span], idx_v)
        for t in range(b_per_w // 16):
            sl = pl.ds(t * 16, 16)
            idx_c[sl] = jnp.clip(idx_v[sl], 0, vocab - 1)
        wi = pltpu.async_copy(idx_v, idx_out.at[row, span], s_wi)

        q = 4
        n_mac = n_ch // q
        ahead = 2 * q

        def start_gather(i):
            m, p = i // q, i % q
            return pltpu.async_copy(
                table_hbm.at[idx_c.at[pl.ds(i * ch, ch)]],
                rows_v.at[m % 3, pl.ds(p * ch, ch)], gsem[i % 9])

        def start_write(m):
            return pltpu.async_copy(
                rows_v.at[m % 3],
                out_hbm.at[row, pl.ds(col + m * q * ch, q * ch)],
                wsem[m % 3])

        gathers = [None] * n_ch
        writes = [None] * n_mac
        for j in range(ahead):
            gathers[j] = start_gather(j)
        for i in range(n_ch):
            j = i + ahead
            if j < n_ch:
                if j % q == 0 and j // q >= 3:
                    writes[j // q - 3].wait()
                gathers[j] = start_gather(j)
            gathers[i].wait()
            if i % q == q - 1:
                writes[i // q] = start_write(i // q)
        for m in range(n_mac - 3, n_mac):
            writes[m].wait()
        wi.wait()

    return gather


def kernel(ids, ids_valid, ids_mask, embed_table):
    vocab, d = embed_table.shape
    b, s = ids.shape
    out, ids_out = _make_gather(b, s, d, vocab)(embed_table, ids)
    return (out, ids_valid, ids_out, ids_mask)


# speedup vs baseline: 1.0196x; 1.0196x over previous
import functools

import jax
import jax.numpy as jnp
from jax import lax
from jax.experimental import pallas as pl
from jax.experimental.pallas import tpu as pltpu
from jax.experimental.pallas import tpu_sc as plsc


def _make_gather(Bb: int, S: int, D: int, vocab: int):
    info = plsc.get_sparse_core_info()
    nw = info.num_cores * info.num_subcores
    b_per_w = (Bb * S) // nw
    assert S % b_per_w == 0
    wpr = S // b_per_w
    ch = 16
    n_ch = b_per_w // ch
    mesh = plsc.VectorSubcoreMesh(core_axis_name="c", subcore_axis_name="s")

    @functools.partial(
        pl.kernel,
        mesh=mesh,
        out_type=(
            jax.ShapeDtypeStruct((Bb, S, D), jnp.float32),
            jax.ShapeDtypeStruct((Bb, S), jnp.int32),
        ),
        scratch_types=[
            pltpu.VMEM((b_per_w,), jnp.int32),
            pltpu.VMEM((b_per_w,), jnp.int32),
            pltpu.VMEM((3, 2 * ch, D), jnp.float32),
        ] + [pltpu.SemaphoreType.DMA] * (6 + 3 + 1),
    )
    def gather(table_hbm, idx_hbm,
               out_hbm, idx_out,
               idx_v, idx_c, rows_v, *sems):
        gsem, wsem = sems[:6], sems[6:9]
        s_wi = sems[9]
        wid = lax.axis_index("s") * info.num_cores + lax.axis_index("c")
        row = wid // wpr
        col = (wid % wpr) * b_per_w
        span = pl.ds(col, b_per_w)

        pltpu.sync_copy(idx_hbm.at[row, span], idx_v)
        for t in range(b_per_w // 16):
            sl = pl.ds(t * 16, 16)
            idx_c[sl] = jnp.clip(idx_v[sl], 0, vocab - 1)
        wi = pltpu.async_copy(idx_v, idx_out.at[row, span], s_wi)

        n_mac = n_ch // 2
        ahead = 4

        def start_gather(i):
            m, p = i // 2, i % 2
            return pltpu.async_copy(
                table_hbm.at[idx_c.at[pl.ds(i * ch, ch)]],
                rows_v.at[m % 3, pl.ds(p * ch, ch)], gsem[i % 6])

        def start_write(m):
            return pltpu.async_copy(
                rows_v.at[m % 3],
                out_hbm.at[row, pl.ds(col + m * 2 * ch, 2 * ch)],
                wsem[m % 3])

        gathers = [None] * n_ch
        writes = [None] * n_mac
        for j in range(ahead):
            gathers[j] = start_gather(j)
        for i in range(n_ch):
            j = i + ahead
            if j < n_ch:
                if j % 2 == 0 and j // 2 >= 3:
                    writes[j // 2 - 3].wait()
                gathers[j] = start_gather(j)
            gathers[i].wait()
            if i % 2 == 1:
                writes[i // 2] = start_write(i // 2)
        for m in range(n_mac - 3, n_mac):
            writes[m].wait()
        wi.wait()

    return gather


def kernel(ids, ids_valid, ids_mask, embed_table):
    vocab, d = embed_table.shape
    b, s = ids.shape
    out, ids_out = _make_gather(b, s, d, vocab)(embed_table, ids)
    return (out, ids_valid, ids_out, ids_mask)
